# initial kernel scaffold (unmeasured)
import jax
import jax.numpy as jnp
from jax import lax
from jax.experimental import pallas as pl
from jax.experimental.pallas import tpu as pltpu

N_DEV = 4
N_TOK = 2048
D_IN = 512
D_OUT = 1024
N_EXP_LOCAL = 8
CAP = 51
CHUNK = N_TOK // N_DEV


def kernel(x, router_W, route_idx, expert_W):
    del router_W

    def body(x_ref, idx_ref, w_ref, out_ref,
             part_ref, sbuf, rbuf, send_sems, recv_sems):
        p = lax.axis_index("i")
        left = lax.rem(p + N_DEV - 1, N_DEV)
        right = lax.rem(p + 1, N_DEV)

        barrier_sem = pltpu.get_barrier_semaphore()
        for nbr in (left, right):
            pl.semaphore_signal(
                barrier_sem, inc=1,
                device_id=(nbr,), device_id_type=pl.DeviceIdType.MESH,
            )
        pl.semaphore_wait(barrier_sem, 2)

        wb = w_ref[:, :, :].astype(jnp.bfloat16)
        tri = (
            lax.broadcasted_iota(jnp.int32, (CHUNK, CHUNK), 0)
            >= lax.broadcasted_iota(jnp.int32, (CHUNK, CHUNK), 1)
        ).astype(jnp.bfloat16)
        eids = lax.broadcasted_iota(jnp.int32, (CHUNK, 32), 1)

        prev = jnp.zeros((1, 32), jnp.float32)
        for c in range(N_DEV):
            rows = pl.ds(c * CHUNK, CHUNK)
            idx_c = idx_ref[rows, :]
            oh = (idx_c == eids).astype(jnp.bfloat16)
            csum = (
                jnp.dot(tri, oh, preferred_element_type=jnp.float32) + prev
            )
            prev = prev + jnp.sum(
                oh.astype(jnp.float32), axis=0, keepdims=True
            )
            sel = jnp.sum(oh * csum, axis=1, keepdims=True)
            keep = sel <= float(CAP)

            x_c = x_ref[rows, :].astype(jnp.bfloat16)
            acc = jnp.zeros((CHUNK, D_OUT), jnp.float32)
            for k in range(N_EXP_LOCAL):
                eg = p * N_EXP_LOCAL + k
                m = jnp.logical_and(idx_c == eg, keep).astype(jnp.bfloat16)
                acc = acc + jnp.dot(
                    x_c * m, wb[k], preferred_element_type=jnp.float32
                )
            part_ref[c] = acc.astype(jnp.bfloat16)

        sbuf[0] = part_ref[p]
        for s in range(N_DEV - 1):
            rdma = pltpu.make_async_remote_copy(
                src_ref=sbuf.at[s],
                dst_ref=rbuf.at[s],
                send_sem=send_sems.at[s],
                recv_sem=recv_sems.at[s],
                device_id=(right,),
                device_id_type=pl.DeviceIdType.MESH,
            )
            rdma.start()
            rdma.wait()
            cr = lax.rem(p + 2 * N_DEV - 1 - s, N_DEV)
            if s < N_DEV - 2:
                sbuf[s + 1] = rbuf[s] + part_ref[cr]
            else:
                red = rbuf[s][...].astype(jnp.float32) + part_ref[cr][
                    ...
                ].astype(jnp.float32)
                out_ref[pl.ds(cr * CHUNK, CHUNK), :] = red
                sbuf[N_DEV - 1] = red.astype(jnp.bfloat16)

        for g in range(N_DEV - 1):
            src = sbuf.at[N_DEV - 1] if g == 0 else rbuf.at[N_DEV - 2 + g]
            t = N_DEV - 1 + g
            rdma = pltpu.make_async_remote_copy(
                src_ref=src,
                dst_ref=rbuf.at[t],
                send_sem=send_sems.at[t],
                recv_sem=recv_sems.at[t],
                device_id=(right,),
                device_id_type=pl.DeviceIdType.MESH,
            )
            rdma.start()
            rdma.wait()
            cidx = lax.rem(p + N_DEV - g, N_DEV)
            out_ref[pl.ds(cidx * CHUNK, CHUNK), :] = rbuf[t][...].astype(
                jnp.float32
            )

    return pl.pallas_call(
        body,
        out_shape=jax.ShapeDtypeStruct((N_TOK, D_OUT), jnp.float32),
        in_specs=[
            pl.BlockSpec(memory_space=pltpu.VMEM),
            pl.BlockSpec(memory_space=pltpu.VMEM),
            pl.BlockSpec(memory_space=pltpu.VMEM),
        ],
        out_specs=pl.BlockSpec(memory_space=pltpu.VMEM),
        scratch_shapes=[
            pltpu.VMEM((N_DEV, CHUNK, D_OUT), jnp.bfloat16),
            pltpu.VMEM((N_DEV, CHUNK, D_OUT), jnp.bfloat16),
            pltpu.VMEM((2 * N_DEV - 2, CHUNK, D_OUT), jnp.bfloat16),
            pltpu.SemaphoreType.DMA((2 * N_DEV - 2,)),
            pltpu.SemaphoreType.DMA((2 * N_DEV - 2,)),
        ],
        compiler_params=pltpu.CompilerParams(collective_id=0),
    )(x, route_idx, expert_W)


# baseline (device time: 121611 ns/iter reference)
import jax
import jax.numpy as jnp
from jax import lax
from jax.experimental import pallas as pl
from jax.experimental.pallas import tpu as pltpu

N_DEV = 4
N_TOK = 2048
D_IN = 512
D_OUT = 1024
N_EXP_LOCAL = 8
CAP = 51
CHUNK = N_TOK // N_DEV


def kernel(x, router_W, route_idx, expert_W):
    del router_W

    def body(x_ref, idx_ref, w_ref, out_ref,
             part_ref, sbuf, rbuf, send_sems, recv_sems):
        p = lax.axis_index("i")
        left = lax.rem(p + N_DEV - 1, N_DEV)
        right = lax.rem(p + 1, N_DEV)

        barrier_sem = pltpu.get_barrier_semaphore()
        for nbr in (left, right):
            pl.semaphore_signal(
                barrier_sem, inc=1,
                device_id=(nbr,), device_id_type=pl.DeviceIdType.MESH,
            )
        pl.semaphore_wait(barrier_sem, 2)

        wb = w_ref[:, :, :].astype(jnp.bfloat16)
        tri = (
            lax.broadcasted_iota(jnp.int32, (CHUNK, CHUNK), 0)
            >= lax.broadcasted_iota(jnp.int32, (CHUNK, CHUNK), 1)
        ).astype(jnp.bfloat16)
        eids = lax.broadcasted_iota(jnp.int32, (CHUNK, 32), 1)

        prev = jnp.zeros((1, 32), jnp.float32)
        for c in range(N_DEV):
            rows = pl.ds(c * CHUNK, CHUNK)
            idx_c = idx_ref[rows, :]
            oh = (idx_c == eids).astype(jnp.bfloat16)
            csum = (
                jnp.dot(tri, oh, preferred_element_type=jnp.float32) + prev
            )
            prev = prev + jnp.sum(
                oh.astype(jnp.float32), axis=0, keepdims=True
            )
            sel = jnp.sum(oh * csum, axis=1, keepdims=True)
            keep = sel <= float(CAP)

            x_c = x_ref[rows, :].astype(jnp.bfloat16)
            acc = jnp.zeros((CHUNK, D_OUT), jnp.float32)
            for k in range(N_EXP_LOCAL):
                eg = p * N_EXP_LOCAL + k
                m = jnp.logical_and(idx_c == eg, keep).astype(jnp.bfloat16)
                acc = acc + jnp.dot(
                    x_c * m, wb[k], preferred_element_type=jnp.float32
                )
            part_ref[c] = acc.astype(jnp.bfloat16)

        sbuf[0] = part_ref[p]
        for s in range(N_DEV - 1):
            rdma = pltpu.make_async_remote_copy(
                src_ref=sbuf.at[s],
                dst_ref=rbuf.at[s],
                send_sem=send_sems.at[s],
                recv_sem=recv_sems.at[s],
                device_id=(right,),
                device_id_type=pl.DeviceIdType.MESH,
            )
            rdma.start()
            rdma.wait()
            cr = lax.rem(p + 2 * N_DEV - 1 - s, N_DEV)
            if s < N_DEV - 2:
                sbuf[s + 1] = rbuf[s] + part_ref[cr]
            else:
                red = rbuf[s][...].astype(jnp.float32) + part_ref[cr][
                    ...
                ].astype(jnp.float32)
                out_ref[pl.ds(cr * CHUNK, CHUNK), :] = red
                sbuf[N_DEV - 1] = red.astype(jnp.bfloat16)

        for g in range(N_DEV - 1):
            src = sbuf.at[N_DEV - 1] if g == 0 else rbuf.at[N_DEV - 2 + g]
            t = N_DEV - 1 + g
            rdma = pltpu.make_async_remote_copy(
                src_ref=src,
                dst_ref=rbuf.at[t],
                send_sem=send_sems.at[t],
                recv_sem=recv_sems.at[t],
                device_id=(right,),
                device_id_type=pl.DeviceIdType.MESH,
            )
            rdma.start()
            rdma.wait()
            cidx = lax.rem(p + N_DEV - g, N_DEV)
            out_ref[pl.ds(cidx * CHUNK, CHUNK), :] = rbuf[t][...].astype(
                jnp.float32
            )

    return pl.pallas_call(
        body,
        out_shape=jax.ShapeDtypeStruct((N_TOK, D_OUT), jnp.float32),
        in_specs=[
            pl.BlockSpec(memory_space=pltpu.VMEM),
            pl.BlockSpec(memory_space=pltpu.VMEM),
            pl.BlockSpec(memory_space=pltpu.VMEM),
        ],
        out_specs=pl.BlockSpec(memory_space=pltpu.VMEM),
        scratch_shapes=[
            pltpu.VMEM((N_DEV, CHUNK, D_OUT), jnp.bfloat16),
            pltpu.VMEM((N_DEV, CHUNK, D_OUT), jnp.bfloat16),
            pltpu.VMEM((2 * N_DEV - 2, CHUNK, D_OUT), jnp.bfloat16),
            pltpu.SemaphoreType.DMA((2 * N_DEV - 2,)),
            pltpu.SemaphoreType.DMA((2 * N_DEV - 2,)),
        ],
        compiler_params=pltpu.CompilerParams(
            collective_id=0,
            vmem_limit_bytes=100 * 1024 * 1024,
        ),
    )(x, route_idx, expert_W)


# device time: 75182 ns/iter; 1.6176x vs baseline; 1.6176x over previous
import jax
import jax.numpy as jnp
from jax import lax
from jax.experimental import pallas as pl
from jax.experimental.pallas import tpu as pltpu

N_DEV = 4
N_TOK = 2048
D_IN = 512
D_OUT = 1024
N_EXP = 32
N_EXP_LOCAL = 8
CAP = 51
CAP_PAD = 64
SLOTS = N_EXP_LOCAL * CAP_PAD
CHUNK = 512


def kernel(x, router_W, route_idx, expert_W):
    del router_W

    def body(x_ref, idx_ref, w_ref, out_ref, ybuf, send_sems, recv_sems):
        p = lax.axis_index("i")
        left = lax.rem(p + N_DEV - 1, N_DEV)
        right = lax.rem(p + 1, N_DEV)

        barrier_sem = pltpu.get_barrier_semaphore()
        for nbr in (left, right):
            pl.semaphore_signal(
                barrier_sem, inc=1,
                device_id=(nbr,), device_id_type=pl.DeviceIdType.MESH,
            )
        pl.semaphore_wait(barrier_sem, 2)

        tri = (
            lax.broadcasted_iota(jnp.int32, (CHUNK, CHUNK), 0)
            >= lax.broadcasted_iota(jnp.int32, (CHUNK, CHUNK), 1)
        ).astype(jnp.bfloat16)
        eids = lax.broadcasted_iota(jnp.int32, (CHUNK, N_EXP), 1)

        prev = jnp.zeros((1, N_EXP), jnp.float32)
        slot_chunks = []
        for c in range(N_TOK // CHUNK):
            idx_c = idx_ref[pl.ds(c * CHUNK, CHUNK), :]
            oh = (idx_c == eids).astype(jnp.bfloat16)
            csum = jnp.dot(tri, oh, preferred_element_type=jnp.float32) + prev
            prev = prev + jnp.sum(oh.astype(jnp.float32), axis=0, keepdims=True)
            sel = jnp.sum(oh * csum, axis=1, keepdims=True)
            keep = sel <= float(CAP)
            rank = sel.astype(jnp.int32) - 1
            slot_chunks.append(
                jnp.where(keep, idx_c * CAP_PAD + rank, -1)
            )
        slot = jnp.concatenate(slot_chunks, axis=0)

        slot_iota = lax.broadcasted_iota(jnp.int32, (N_TOK, SLOTS), 1)

        d_mine = ((slot - p * SLOTS) == slot_iota).astype(jnp.bfloat16)
        xb = x_ref[:, :].astype(jnp.bfloat16)
        xg = lax.dot_general(
            d_mine, xb, (((0,), (0,)), ((), ())),
            preferred_element_type=jnp.float32,
        ).astype(jnp.bfloat16)
        for k in range(N_EXP_LOCAL):
            wk = w_ref[k].astype(jnp.bfloat16)
            ybuf[0, pl.ds(k * CAP_PAD, CAP_PAD), :] = jnp.dot(
                xg[k * CAP_PAD:(k + 1) * CAP_PAD, :], wk,
                preferred_element_type=jnp.float32,
            ).astype(jnp.bfloat16)

        out_ref[:, :] = jnp.dot(
            d_mine, ybuf[0], preferred_element_type=jnp.float32
        )

        for h in range(N_DEV - 1):
            rdma = pltpu.make_async_remote_copy(
                src_ref=ybuf.at[h],
                dst_ref=ybuf.at[h + 1],
                send_sem=send_sems.at[h],
                recv_sem=recv_sems.at[h],
                device_id=(right,),
                device_id_type=pl.DeviceIdType.MESH,
            )
            rdma.start()
            rdma.wait()
            q = lax.rem(p + 2 * N_DEV - 1 - h, N_DEV)
            d_q = ((slot - q * SLOTS) == slot_iota).astype(jnp.bfloat16)
            out_ref[:, :] = out_ref[:, :] + jnp.dot(
                d_q, ybuf[h + 1], preferred_element_type=jnp.float32
            )

    return pl.pallas_call(
        body,
        out_shape=jax.ShapeDtypeStruct((N_TOK, D_OUT), jnp.float32),
        in_specs=[
            pl.BlockSpec(memory_space=pltpu.VMEM),
            pl.BlockSpec(memory_space=pltpu.VMEM),
            pl.BlockSpec(memory_space=pltpu.VMEM),
        ],
        out_specs=pl.BlockSpec(memory_space=pltpu.VMEM),
        scratch_shapes=[
            pltpu.VMEM((N_DEV, SLOTS, D_OUT), jnp.bfloat16),
            pltpu.SemaphoreType.DMA((N_DEV - 1,)),
            pltpu.SemaphoreType.DMA((N_DEV - 1,)),
        ],
        compiler_params=pltpu.CompilerParams(
            collective_id=0,
            vmem_limit_bytes=100 * 1024 * 1024,
        ),
    )(x, route_idx, expert_W)


# device time: 55083 ns/iter; 2.2078x vs baseline; 1.3649x over previous
import jax
import jax.numpy as jnp
from jax import lax
from jax.experimental import pallas as pl
from jax.experimental.pallas import tpu as pltpu

N_DEV = 4
N_TOK = 2048
D_IN = 512
D_OUT = 1024
N_EXP = 32
N_EXP_LOCAL = 8
CAP = 51
CAP_PAD = 64
SLOTS = N_EXP_LOCAL * CAP_PAD
CHUNK = 512


def kernel(x, router_W, route_idx, expert_W):
    del router_W

    def body(x_ref, idx_ref, w_ref, out_ref, ybuf, send_sems, recv_sems):
        p = lax.axis_index("i")
        left = lax.rem(p + N_DEV - 1, N_DEV)
        right = lax.rem(p + 1, N_DEV)

        diag = lax.rem(p + 2, N_DEV)

        barrier_sem = pltpu.get_barrier_semaphore()
        for nbr in (left, right, diag):
            pl.semaphore_signal(
                barrier_sem, inc=1,
                device_id=(nbr,), device_id_type=pl.DeviceIdType.MESH,
            )
        pl.semaphore_wait(barrier_sem, 3)

        tri = (
            lax.broadcasted_iota(jnp.int32, (CHUNK, CHUNK), 0)
            >= lax.broadcasted_iota(jnp.int32, (CHUNK, CHUNK), 1)
        ).astype(jnp.bfloat16)
        eids = lax.broadcasted_iota(jnp.int32, (CHUNK, N_EXP), 1)

        prev = jnp.zeros((1, N_EXP), jnp.float32)
        slot_chunks = []
        for c in range(N_TOK // CHUNK):
            idx_c = idx_ref[pl.ds(c * CHUNK, CHUNK), :]
            oh = (idx_c == eids).astype(jnp.bfloat16)
            csum = jnp.dot(tri, oh, preferred_element_type=jnp.float32) + prev
            prev = prev + jnp.sum(oh.astype(jnp.float32), axis=0, keepdims=True)
            sel = jnp.sum(oh * csum, axis=1, keepdims=True)
            keep = sel <= float(CAP)
            rank = sel.astype(jnp.int32) - 1
            slot_chunks.append(
                jnp.where(keep, idx_c * CAP_PAD + rank, -1)
            )
        slot = jnp.concatenate(slot_chunks, axis=0)

        slot_iota = lax.broadcasted_iota(jnp.int32, (N_TOK, SLOTS), 1)

        d_mine = ((slot - p * SLOTS) == slot_iota).astype(jnp.bfloat16)
        xb = x_ref[:, :].astype(jnp.bfloat16)
        xg = lax.dot_general(
            d_mine, xb, (((0,), (0,)), ((), ())),
            preferred_element_type=jnp.float32,
        ).astype(jnp.bfloat16)
        for k in range(N_EXP_LOCAL):
            wk = w_ref[k].astype(jnp.bfloat16)
            ybuf[0, pl.ds(k * CAP_PAD, CAP_PAD), :] = jnp.dot(
                xg[k * CAP_PAD:(k + 1) * CAP_PAD, :], wk,
                preferred_element_type=jnp.float32,
            ).astype(jnp.bfloat16)

        sends = []
        for i, (tgt, slot_id) in enumerate(((left, 1), (right, 2), (diag, 3))):
            rdma = pltpu.make_async_remote_copy(
                src_ref=ybuf.at[0],
                dst_ref=ybuf.at[slot_id],
                send_sem=send_sems.at[i],
                recv_sem=recv_sems.at[slot_id - 1],
                device_id=(tgt,),
                device_id_type=pl.DeviceIdType.MESH,
            )
            rdma.start()
            sends.append(rdma)

        out_ref[:, :] = jnp.dot(
            d_mine, ybuf[0], preferred_element_type=jnp.float32
        )

        for slot_id, q in ((1, right), (2, left), (3, diag)):
            d_q = ((slot - q * SLOTS) == slot_iota).astype(jnp.bfloat16)
            recv = pltpu.make_async_remote_copy(
                src_ref=ybuf.at[0],
                dst_ref=ybuf.at[slot_id],
                send_sem=send_sems.at[0],
                recv_sem=recv_sems.at[slot_id - 1],
                device_id=(q,),
                device_id_type=pl.DeviceIdType.MESH,
            )
            recv.wait_recv()
            out_ref[:, :] = out_ref[:, :] + jnp.dot(
                d_q, ybuf[slot_id], preferred_element_type=jnp.float32
            )

        for rdma in sends:
            rdma.wait_send()

    return pl.pallas_call(
        body,
        out_shape=jax.ShapeDtypeStruct((N_TOK, D_OUT), jnp.float32),
        in_specs=[
            pl.BlockSpec(memory_space=pltpu.VMEM),
            pl.BlockSpec(memory_space=pltpu.VMEM),
            pl.BlockSpec(memory_space=pltpu.VMEM),
        ],
        out_specs=pl.BlockSpec(memory_space=pltpu.VMEM),
        scratch_shapes=[
            pltpu.VMEM((N_DEV, SLOTS, D_OUT), jnp.bfloat16),
            pltpu.SemaphoreType.DMA((N_DEV - 1,)),
            pltpu.SemaphoreType.DMA((N_DEV - 1,)),
        ],
        compiler_params=pltpu.CompilerParams(
            collective_id=0,
            vmem_limit_bytes=100 * 1024 * 1024,
        ),
    )(x, route_idx, expert_W)
